# baseline (speedup 1.0000x reference)
import functools
import math

import jax
import jax.numpy as jnp
from jax.experimental import pallas as pl
from jax.experimental.pallas import tpu as pltpu

_TEMP = 0.5
_INV_SQRT2 = 1.0 / math.sqrt(2.0)


def _ffn_kernel(x_ref, sig_ref, wup_ref, bup_ref, wdown_ref, bdown_ref,
                wcls_ref, bcls_ref, logits_ref, idx_ref, w_ref, acc_ref,
                *, n_h):
    t = pl.program_id(1)
    h = pl.program_id(2)
    n_t = pl.num_programs(1)

    x = x_ref[...]
    x_bf = x.astype(jnp.bfloat16)

    @pl.when(jnp.logical_and(t == 0, h == 0))
    def _init():
        scores = jax.lax.dot_general(
            x, sig_ref[...], (((1,), (1,)), ((), ())),
            preferred_element_type=jnp.float32)
        idx_ref[...] = jnp.argmax(scores, axis=-1).astype(jnp.int32)[:, None]
        z = scores * (1.0 / _TEMP)
        z = z - jnp.max(z, axis=-1, keepdims=True)
        e = jnp.exp(z)
        w = e / jnp.sum(e, axis=-1, keepdims=True)
        w_ref[...] = w
        acc_ref[...] = x + jax.lax.dot_general(
            w, bdown_ref[...], (((1,), (0,)), ((), ())),
            preferred_element_type=jnp.float32)

    hpre = jax.lax.dot_general(
        x_bf, wup_ref[0], (((1,), (1,)), ((), ())),
        preferred_element_type=jnp.float32) + bup_ref[0]
    hact = 0.5 * hpre * (1.0 + jax.lax.erf(hpre * _INV_SQRT2))
    onehot = (jax.lax.broadcasted_iota(jnp.int32, (1, w_ref.shape[1]), 1) == t)
    wcol = jnp.sum(jnp.where(onehot, w_ref[...], 0.0), axis=1, keepdims=True)
    acc_ref[...] += jax.lax.dot_general(
        (hact * wcol).astype(jnp.bfloat16), wdown_ref[0],
        (((1,), (1,)), ((), ())), preferred_element_type=jnp.float32)

    @pl.when(jnp.logical_and(t == n_t - 1, h == n_h - 1))
    def _classify():
        logits_ref[...] = jax.lax.dot_general(
            acc_ref[...].astype(jnp.bfloat16), wcls_ref[...],
            (((1,), (1,)), ((), ())),
            preferred_element_type=jnp.float32) + bcls_ref[...]


def kernel(x, signatures, W_up, b_up, W_down, b_down, W_cls, b_cls,
           *, interpret=False):
    B, D = x.shape
    T, H, _ = W_up.shape
    C = W_cls.shape[0]
    BB = min(1024, B)
    BH = min(1024, H)
    n_b, n_h = B // BB, H // BH
    Cp = ((C + 127) // 128) * 128
    W_cls_p = jnp.pad(W_cls, ((0, Cp - C), (0, 0))).astype(jnp.bfloat16)
    b_cls_p = jnp.pad(b_cls, (0, Cp - C)).reshape(1, Cp)

    logits_p, idx2, weights = pl.pallas_call(
        functools.partial(_ffn_kernel, n_h=n_h),
        grid=(n_b, T, n_h),
        in_specs=[
            pl.BlockSpec((BB, D), lambda i, t, h: (i, 0)),
            pl.BlockSpec((T, D), lambda i, t, h: (0, 0)),
            pl.BlockSpec((1, BH, D), lambda i, t, h: (t, h, 0)),
            pl.BlockSpec((1, 1, BH), lambda i, t, h: (t, 0, h)),
            pl.BlockSpec((1, D, BH), lambda i, t, h: (t, 0, h)),
            pl.BlockSpec((T, D), lambda i, t, h: (0, 0)),
            pl.BlockSpec((Cp, D), lambda i, t, h: (0, 0)),
            pl.BlockSpec((1, Cp), lambda i, t, h: (0, 0)),
        ],
        out_specs=[
            pl.BlockSpec((BB, Cp), lambda i, t, h: (i, 0)),
            pl.BlockSpec((BB, 1), lambda i, t, h: (i, 0)),
            pl.BlockSpec((BB, T), lambda i, t, h: (i, 0)),
        ],
        out_shape=[
            jax.ShapeDtypeStruct((B, Cp), jnp.float32),
            jax.ShapeDtypeStruct((B, 1), jnp.int32),
            jax.ShapeDtypeStruct((B, T), jnp.float32),
        ],
        scratch_shapes=[pltpu.VMEM((BB, D), jnp.float32)],
        compiler_params=pltpu.CompilerParams(
            dimension_semantics=("parallel", "arbitrary", "arbitrary"),
            vmem_limit_bytes=48 * 1024 * 1024,
        ),
        name="trix_ffn",
        interpret=interpret,
    )(x, signatures, W_up.astype(jnp.bfloat16), b_up.reshape(T, 1, H),
      W_down.astype(jnp.bfloat16), b_down, W_cls_p, b_cls_p)
    return logits_p[:, :C], idx2[:, 0], weights


# speedup vs baseline: 1.2327x; 1.2327x over previous
import functools
import math

import jax
import jax.numpy as jnp
from jax.experimental import pallas as pl
from jax.experimental.pallas import tpu as pltpu

_TEMP = 0.5
_INV_SQRT2 = 1.0 / math.sqrt(2.0)


def _ffn_kernel(x_ref, sig_ref, wup_ref, bup_ref, wdown_ref, bdown_ref,
                wcls_ref, bcls_ref, logits_ref, idx_ref, w_ref, acc_ref,
                *, n_h):
    t = pl.program_id(1)
    h = pl.program_id(2)
    n_t = pl.num_programs(1)

    x = x_ref[...]

    @pl.when(jnp.logical_and(t == 0, h == 0))
    def _init():
        scores = jax.lax.dot_general(
            x, sig_ref[...], (((1,), (1,)), ((), ())),
            preferred_element_type=jnp.float32)
        idx_ref[...] = jnp.argmax(scores, axis=-1).astype(jnp.int32)[:, None]
        z = scores * (1.0 / _TEMP)
        z = z - jnp.max(z, axis=-1, keepdims=True)
        e = jnp.exp(z)
        w = e / jnp.sum(e, axis=-1, keepdims=True)
        w_ref[...] = w
        acc_ref[...] = x + jax.lax.dot_general(
            w, bdown_ref[...], (((1,), (0,)), ((), ())),
            preferred_element_type=jnp.float32)

    hpre = jax.lax.dot_general(
        x, wup_ref[0], (((1,), (1,)), ((), ())),
        preferred_element_type=jnp.float32) + bup_ref[0]
    hact = 0.5 * hpre * (1.0 + jax.lax.erf(hpre * _INV_SQRT2))
    onehot = (jax.lax.broadcasted_iota(jnp.int32, (1, w_ref.shape[1]), 1) == t)
    wcol = jnp.sum(jnp.where(onehot, w_ref[...], 0.0), axis=1, keepdims=True)
    acc_ref[...] += jax.lax.dot_general(
        hact * wcol, wdown_ref[0],
        (((1,), (1,)), ((), ())), preferred_element_type=jnp.float32)

    @pl.when(jnp.logical_and(t == n_t - 1, h == n_h - 1))
    def _classify():
        logits_ref[...] = jax.lax.dot_general(
            acc_ref[...], wcls_ref[...], (((1,), (1,)), ((), ())),
            preferred_element_type=jnp.float32) + bcls_ref[...]


def kernel(x, signatures, W_up, b_up, W_down, b_down, W_cls, b_cls,
           *, interpret=False):
    B, D = x.shape
    T, H, _ = W_up.shape
    C = W_cls.shape[0]
    BB = min(1024, B)
    BH = min(1024, H)
    n_b, n_h = B // BB, H // BH
    Cp = ((C + 127) // 128) * 128
    W_cls_p = jnp.pad(W_cls, ((0, Cp - C), (0, 0)))
    b_cls_p = jnp.pad(b_cls, (0, Cp - C)).reshape(1, Cp)

    logits_p, idx2, weights = pl.pallas_call(
        functools.partial(_ffn_kernel, n_h=n_h),
        grid=(n_b, T, n_h),
        in_specs=[
            pl.BlockSpec((BB, D), lambda i, t, h: (i, 0)),
            pl.BlockSpec((T, D), lambda i, t, h: (0, 0)),
            pl.BlockSpec((1, BH, D), lambda i, t, h: (t, h, 0)),
            pl.BlockSpec((1, 1, BH), lambda i, t, h: (t, 0, h)),
            pl.BlockSpec((1, D, BH), lambda i, t, h: (t, 0, h)),
            pl.BlockSpec((T, D), lambda i, t, h: (0, 0)),
            pl.BlockSpec((Cp, D), lambda i, t, h: (0, 0)),
            pl.BlockSpec((1, Cp), lambda i, t, h: (0, 0)),
        ],
        out_specs=[
            pl.BlockSpec((BB, Cp), lambda i, t, h: (i, 0)),
            pl.BlockSpec((BB, 1), lambda i, t, h: (i, 0)),
            pl.BlockSpec((BB, T), lambda i, t, h: (i, 0)),
        ],
        out_shape=[
            jax.ShapeDtypeStruct((B, Cp), jnp.float32),
            jax.ShapeDtypeStruct((B, 1), jnp.int32),
            jax.ShapeDtypeStruct((B, T), jnp.float32),
        ],
        scratch_shapes=[pltpu.VMEM((BB, D), jnp.float32)],
        compiler_params=pltpu.CompilerParams(
            dimension_semantics=("parallel", "arbitrary", "arbitrary"),
            vmem_limit_bytes=48 * 1024 * 1024,
        ),
        name="trix_ffn",
        interpret=interpret,
    )(x, signatures, W_up, b_up.reshape(T, 1, H), W_down, b_down,
      W_cls_p, b_cls_p)
    return logits_p[:, :C], idx2[:, 0], weights


# speedup vs baseline: 1.2373x; 1.0037x over previous
import functools
import math

import jax
import jax.numpy as jnp
from jax.experimental import pallas as pl
from jax.experimental.pallas import tpu as pltpu

_TEMP = 0.5
_INV_SQRT2 = 1.0 / math.sqrt(2.0)


def _ffn_kernel(x_ref, sig_ref, wup_ref, bup_ref, wdown_ref, bdown_ref,
                out_ref, idx_ref, w_ref):
    t = pl.program_id(1)
    h = pl.program_id(2)

    x = x_ref[...]

    @pl.when(jnp.logical_and(t == 0, h == 0))
    def _init():
        scores = jax.lax.dot_general(
            x, sig_ref[...], (((1,), (1,)), ((), ())),
            preferred_element_type=jnp.float32)
        idx_ref[...] = jnp.argmax(scores, axis=-1).astype(jnp.int32)[:, None]
        z = scores * (1.0 / _TEMP)
        z = z - jnp.max(z, axis=-1, keepdims=True)
        e = jnp.exp(z)
        w = e / jnp.sum(e, axis=-1, keepdims=True)
        w_ref[...] = w
        out_ref[...] = x + jax.lax.dot_general(
            w, bdown_ref[...], (((1,), (0,)), ((), ())),
            preferred_element_type=jnp.float32)

    hpre = jax.lax.dot_general(
        x, wup_ref[0], (((1,), (1,)), ((), ())),
        preferred_element_type=jnp.float32) + bup_ref[0]
    hact = 0.5 * hpre * (1.0 + jax.lax.erf(hpre * _INV_SQRT2))
    onehot = (jax.lax.broadcasted_iota(jnp.int32, (1, w_ref.shape[1]), 1) == t)
    wcol = jnp.sum(jnp.where(onehot, w_ref[...], 0.0), axis=1, keepdims=True)
    out_ref[...] += wcol * jax.lax.dot_general(
        hact, wdown_ref[0], (((1,), (1,)), ((), ())),
        preferred_element_type=jnp.float32)


def _cls_kernel(o_ref, wcls_ref, bcls_ref, logits_ref):
    logits_ref[...] = jax.lax.dot_general(
        o_ref[...], wcls_ref[...], (((1,), (1,)), ((), ())),
        preferred_element_type=jnp.float32) + bcls_ref[...]


def kernel(x, signatures, W_up, b_up, W_down, b_down, W_cls, b_cls,
           *, interpret=False):
    B, D = x.shape
    T, H, _ = W_up.shape
    C = W_cls.shape[0]
    BB = min(1024, B)
    BH = min(1024, H)
    n_b, n_h = B // BB, H // BH
    Cp = ((C + 127) // 128) * 128
    W_cls_p = jnp.pad(W_cls, ((0, Cp - C), (0, 0)))
    b_cls_p = jnp.pad(b_cls, (0, Cp - C)).reshape(1, Cp)

    outputs, idx2, weights = pl.pallas_call(
        _ffn_kernel,
        grid=(n_b, T, n_h),
        in_specs=[
            pl.BlockSpec((BB, D), lambda i, t, h: (i, 0)),
            pl.BlockSpec((T, D), lambda i, t, h: (0, 0)),
            pl.BlockSpec((1, BH, D), lambda i, t, h: (t, h, 0)),
            pl.BlockSpec((1, 1, BH), lambda i, t, h: (t, 0, h)),
            pl.BlockSpec((1, D, BH), lambda i, t, h: (t, 0, h)),
            pl.BlockSpec((T, D), lambda i, t, h: (0, 0)),
        ],
        out_specs=[
            pl.BlockSpec((BB, D), lambda i, t, h: (i, 0)),
            pl.BlockSpec((BB, 1), lambda i, t, h: (i, 0)),
            pl.BlockSpec((BB, T), lambda i, t, h: (i, 0)),
        ],
        out_shape=[
            jax.ShapeDtypeStruct((B, D), jnp.float32),
            jax.ShapeDtypeStruct((B, 1), jnp.int32),
            jax.ShapeDtypeStruct((B, T), jnp.float32),
        ],
        compiler_params=pltpu.CompilerParams(
            dimension_semantics=("parallel", "arbitrary", "arbitrary"),
            vmem_limit_bytes=54 * 1024 * 1024,
        ),
        name="trix_ffn",
        interpret=interpret,
    )(x, signatures, W_up, b_up.reshape(T, 1, H), W_down, b_down)

    BBc = min(1024, B)
    logits_p = pl.pallas_call(
        _cls_kernel,
        grid=(B // BBc,),
        in_specs=[
            pl.BlockSpec((BBc, D), lambda i: (i, 0)),
            pl.BlockSpec((Cp, D), lambda i: (0, 0)),
            pl.BlockSpec((1, Cp), lambda i: (0, 0)),
        ],
        out_specs=pl.BlockSpec((BBc, Cp), lambda i: (i, 0)),
        out_shape=jax.ShapeDtypeStruct((B, Cp), jnp.float32),
        compiler_params=pltpu.CompilerParams(
            dimension_semantics=("parallel",),
            vmem_limit_bytes=40 * 1024 * 1024,
        ),
        name="trix_cls",
        interpret=interpret,
    )(outputs, W_cls_p, b_cls_p)
    return logits_p[:, :C], idx2[:, 0], weights


# speedup vs baseline: 1.2632x; 1.0210x over previous
import functools
import math

import jax
import jax.numpy as jnp
from jax.experimental import pallas as pl
from jax.experimental.pallas import tpu as pltpu

_TEMP = 0.5
_INV_SQRT2 = 1.0 / math.sqrt(2.0)


def _ffn_kernel(x_ref, sig_ref, wup_ref, bup_ref, wdown_ref, bdown_ref,
                out_ref, idx_ref, w_ref):
    t = pl.program_id(1)
    h = pl.program_id(2)

    x = x_ref[...]

    @pl.when(jnp.logical_and(t == 0, h == 0))
    def _init():
        scores = jax.lax.dot_general(
            x, sig_ref[...], (((1,), (1,)), ((), ())),
            preferred_element_type=jnp.float32)
        idx_ref[...] = jnp.argmax(scores, axis=-1).astype(jnp.int32)[:, None]
        z = scores * (1.0 / _TEMP)
        z = z - jnp.max(z, axis=-1, keepdims=True)
        e = jnp.exp(z)
        w = e / jnp.sum(e, axis=-1, keepdims=True)
        w_ref[...] = w
        out_ref[...] = x + jax.lax.dot_general(
            w, bdown_ref[...], (((1,), (0,)), ((), ())),
            preferred_element_type=jnp.float32)

    onehot = (jax.lax.broadcasted_iota(jnp.int32, (1, w_ref.shape[1]), 1) == t)
    wcol = jnp.sum(jnp.where(onehot, w_ref[...], 0.0), axis=1, keepdims=True)
    hpre = jax.lax.dot_general(
        x, wup_ref[0], (((1,), (1,)), ((), ())),
        preferred_element_type=jnp.float32) + bup_ref[0]
    hact = 0.5 * hpre * (1.0 + jax.lax.erf(hpre * _INV_SQRT2))
    out_ref[...] += wcol * jax.lax.dot_general(
        hact, wdown_ref[0], (((1,), (1,)), ((), ())),
        preferred_element_type=jnp.float32)


def _cls_kernel(o_ref, wcls_ref, bcls_ref, logits_ref):
    logits_ref[...] = jax.lax.dot_general(
        o_ref[...], wcls_ref[...], (((1,), (1,)), ((), ())),
        preferred_element_type=jnp.float32) + bcls_ref[...]


def kernel(x, signatures, W_up, b_up, W_down, b_down, W_cls, b_cls,
           *, interpret=False):
    B, D = x.shape
    T, H, _ = W_up.shape
    C = W_cls.shape[0]
    BB = min(1024, B)
    BH = min(1024, H)
    n_b, n_h = B // BB, H // BH
    b_cls_2d = b_cls.reshape(1, C)

    outputs, idx2, weights = pl.pallas_call(
        _ffn_kernel,
        grid=(n_b, T, n_h),
        in_specs=[
            pl.BlockSpec((BB, D), lambda i, t, h: (i, 0)),
            pl.BlockSpec((T, D), lambda i, t, h: (0, 0)),
            pl.BlockSpec((1, BH, D), lambda i, t, h: (t, h, 0)),
            pl.BlockSpec((1, 1, BH), lambda i, t, h: (t, 0, h)),
            pl.BlockSpec((1, D, BH), lambda i, t, h: (t, 0, h)),
            pl.BlockSpec((T, D), lambda i, t, h: (0, 0)),
        ],
        out_specs=[
            pl.BlockSpec((BB, D), lambda i, t, h: (i, 0)),
            pl.BlockSpec((BB, 1), lambda i, t, h: (i, 0)),
            pl.BlockSpec((BB, T), lambda i, t, h: (i, 0)),
        ],
        out_shape=[
            jax.ShapeDtypeStruct((B, D), jnp.float32),
            jax.ShapeDtypeStruct((B, 1), jnp.int32),
            jax.ShapeDtypeStruct((B, T), jnp.float32),
        ],
        compiler_params=pltpu.CompilerParams(
            dimension_semantics=("parallel", "arbitrary", "arbitrary"),
            vmem_limit_bytes=54 * 1024 * 1024,
        ),
        name="trix_ffn",
        interpret=interpret,
    )(x, signatures, W_up, b_up.reshape(T, 1, H), W_down, b_down)

    BBc = min(1024, B)
    logits = pl.pallas_call(
        _cls_kernel,
        grid=(B // BBc,),
        in_specs=[
            pl.BlockSpec((BBc, D), lambda i: (i, 0)),
            pl.BlockSpec((C, D), lambda i: (0, 0)),
            pl.BlockSpec((1, C), lambda i: (0, 0)),
        ],
        out_specs=pl.BlockSpec((BBc, C), lambda i: (i, 0)),
        out_shape=jax.ShapeDtypeStruct((B, C), jnp.float32),
        compiler_params=pltpu.CompilerParams(
            dimension_semantics=("parallel",),
            vmem_limit_bytes=40 * 1024 * 1024,
        ),
        name="trix_cls",
        interpret=interpret,
    )(outputs, W_cls, b_cls_2d)
    return logits, idx2[:, 0], weights
